# Initial kernel scaffold; baseline (speedup 1.0000x reference)
import jax
import jax.numpy as jnp
from jax.experimental import pallas as pl


def kernel(node_features, edge_features, senders, receivers, enc_edge_W, enc_edge_b, enc_node_W, enc_node_b, hid_edge_We, hid_edge_Ws, hid_edge_b, hid_node_Wn, hid_node_Wi, hid_node_b, ro_node_W, ro_node_b, ro_glob_W, ro_glob_b):
    raise NotImplementedError("write your pallas kernel here")


# speedup vs baseline: 4.0595x; 4.0595x over previous
import functools

import jax
import jax.numpy as jnp
from jax import lax
from jax.experimental import pallas as pl
from jax.experimental.pallas import tpu as pltpu
from jax.experimental.pallas import tpu_sc as plsc

_NC = 2
_NS = 16
_NW = _NC * _NS
_LANES = 16


def _bf(x):
    return x.astype(jnp.bfloat16).astype(jnp.float32)


def _bmm(x, w_ref, k):
    acc = _bf(x[:, 0:1]) * _bf(w_ref[0:1, :])
    for i in range(1, k):
        acc = acc + _bf(x[:, i : i + 1]) * _bf(w_ref[i : i + 1, :])
    return acc


def _node_enc_kernel(nf_ref, wn_ref, bn_ref, ws_ref, be_ref, n_ref, m_ref):
    nf = nf_ref[...]
    n = jnp.maximum(_bmm(nf, wn_ref, 4) + bn_ref[...], 0.0)
    n_ref[...] = n
    m_ref[...] = _bmm(n, ws_ref, 8) + be_ref[...]


def _edge_enc_kernel(ef_ref, w1_ref, b1_ref, we_ref, t_ref):
    ef = ef_ref[...]
    e = jnp.maximum(_bmm(ef, w1_ref, 2) + b1_ref[...], 0.0)
    t_ref[...] = _bmm(e, we_ref, 4)


def _node_out_kernel(n_ref, p0_ref, p1_ref, wn_ref, wi_ref, b_ref,
                     wro_ref, bro_ref, wg_ref, bg_ref,
                     nodes_ref, glob_ref, gacc_ref):
    i = pl.program_id(0)
    nb = pl.num_programs(0)
    n = n_ref[...]
    acc = p0_ref[...] + p1_ref[...]
    n2 = jnp.maximum(_bmm(n, wn_ref, 8) + _bmm(acc, wi_ref, 8) + b_ref[...], 0.0)
    nodes_ref[...] = _bmm(n2, wro_ref, 8) + bro_ref[...]
    bsum = jnp.sum(n2, axis=0, keepdims=True)

    @pl.when(i == 0)
    def _():
        gacc_ref[...] = bsum

    @pl.when(i > 0)
    def _():
        gacc_ref[...] = gacc_ref[...] + bsum

    @pl.when(i == nb - 1)
    def _():
        glob_ref[...] = _bmm(gacc_ref[...], wg_ref, 8) + bg_ref[...]


def _sc_edge_body(n_nodes, n_groups, t_hbm, s_hbm, r_hbm, m_hbm, z_hbm,
                  out_hbm, acc_sp, s_v, r_v, t_v, mg_v, stage_v, sem):
    cid = lax.axis_index("c")
    sid = lax.axis_index("s")
    wid = sid * _NC + cid

    sr = 800
    nb = (n_nodes // _NS + sr - 1) // sr
    span = sr * nb
    base = pl.multiple_of(jnp.minimum(sid * span, n_nodes - span), 8)

    pltpu.sync_copy(z_hbm.at[pl.ds(0, sr)], stage_v)

    def stage_z(k, c):
        r0 = pl.multiple_of(base + k * sr, 8)
        pltpu.sync_copy(stage_v, acc_sp.at[pl.ds(r0, sr)])
        return c

    lax.fori_loop(0, nb, stage_z, 0)
    plsc.subcore_barrier()

    gper = n_groups // _NW
    rem = n_groups - gper * _NW
    ngroups = gper + jnp.where(wid < rem, 1, 0)
    g0 = wid * gper + jnp.minimum(wid, rem)

    iota = lax.iota(jnp.int32, _LANES)
    col16 = iota % 8
    rowadd = iota // 8

    def group_body(i, carry):
        g = g0 + i
        pltpu.sync_copy(s_hbm.at[pl.ds(g * 128, 128)], s_v)
        pltpu.sync_copy(r_hbm.at[pl.ds(g * 128, 128)], r_v)
        pltpu.sync_copy(t_hbm.at[pl.ds(g * 1024, 1024)], t_v)
        pltpu.async_copy(m_hbm.at[s_v], mg_v, sem).wait()

        def compute_body(j, c):
            row16 = 2 * j + rowadd
            tv = t_v[pl.ds(j * 16, 16)]
            mv = plsc.load_gather(mg_v, [row16, col16])
            e2 = jnp.maximum(tv + mv, 0.0)
            plsc.store_scatter(mg_v, [row16, col16], e2)
            return c

        lax.fori_loop(0, 64, compute_body, 0)
        pltpu.sync_copy(mg_v, acc_sp.at[r_v], add=True)
        return carry

    lax.fori_loop(0, ngroups, group_body, 0)
    plsc.subcore_barrier()

    def unstage(k, c):
        r0 = pl.multiple_of(base + k * sr, 8)
        pltpu.sync_copy(acc_sp.at[pl.ds(r0, sr)], stage_v)
        pltpu.sync_copy(stage_v,
                        out_hbm.at[pl.ds(cid * n_nodes + r0, sr)])
        return c

    lax.fori_loop(0, nb, unstage, 0)


def _sc_edge_call(n_nodes, n_groups, t_edge, senders, receivers, m_tab, zeros):
    mesh = plsc.VectorSubcoreMesh(core_axis_name="c", subcore_axis_name="s",
                                  num_cores=_NC, num_subcores=_NS)
    return pl.kernel(
        functools.partial(_sc_edge_body, n_nodes, n_groups),
        out_type=jax.ShapeDtypeStruct((_NC * n_nodes, 8), jnp.float32),
        mesh=mesh,
        scratch_types=[
            pltpu.VMEM_SHARED((n_nodes, 8), jnp.float32),
            pltpu.VMEM((128,), jnp.int32),
            pltpu.VMEM((128,), jnp.int32),
            pltpu.VMEM((1024,), jnp.float32),
            pltpu.VMEM((128, 8), jnp.float32),
            pltpu.VMEM((800, 8), jnp.float32),
            pltpu.SemaphoreType.DMA,
        ],
        compiler_params=pltpu.CompilerParams(
            needs_layout_passes=False, use_tc_tiling_on_sc=False),
    )(t_edge, senders, receivers, m_tab, zeros)


def kernel(node_features, edge_features, senders, receivers, enc_edge_W,
           enc_edge_b, enc_node_W, enc_node_b, hid_edge_We, hid_edge_Ws,
           hid_edge_b, hid_node_Wn, hid_node_Wi, hid_node_b, ro_node_W,
           ro_node_b, ro_glob_W, ro_glob_b):
    n_nodes = node_features.shape[0]
    n_edges = edge_features.shape[0]
    assert n_edges % 128 == 0
    n_groups = n_edges // 128

    f32 = jnp.float32

    bn = 4000 if n_nodes % 4000 == 0 else n_nodes
    n_blocks = n_nodes // bn
    n_enc, m_tab = pl.pallas_call(
        _node_enc_kernel,
        grid=(n_blocks,),
        in_specs=[
            pl.BlockSpec((bn, 4), lambda i: (i, 0)),
            pl.BlockSpec((4, 8), lambda i: (0, 0)),
            pl.BlockSpec((1, 8), lambda i: (0, 0)),
            pl.BlockSpec((8, 8), lambda i: (0, 0)),
            pl.BlockSpec((1, 8), lambda i: (0, 0)),
        ],
        out_specs=[
            pl.BlockSpec((bn, 8), lambda i: (i, 0)),
            pl.BlockSpec((bn, 8), lambda i: (i, 0)),
        ],
        out_shape=[
            jax.ShapeDtypeStruct((n_nodes, 8), f32),
            jax.ShapeDtypeStruct((n_nodes, 8), f32),
        ],
    )(node_features, enc_node_W, enc_node_b.reshape(1, 8), hid_edge_Ws,
      hid_edge_b.reshape(1, 8))

    be = 6400 if n_edges % 6400 == 0 else n_edges
    e_blocks = n_edges // be
    t_edge = pl.pallas_call(
        _edge_enc_kernel,
        grid=(e_blocks,),
        in_specs=[
            pl.BlockSpec((be, 2), lambda i: (i, 0)),
            pl.BlockSpec((2, 4), lambda i: (0, 0)),
            pl.BlockSpec((1, 4), lambda i: (0, 0)),
            pl.BlockSpec((4, 8), lambda i: (0, 0)),
        ],
        out_specs=pl.BlockSpec((be, 8), lambda i: (i, 0)),
        out_shape=jax.ShapeDtypeStruct((n_edges, 8), f32),
    )(edge_features, enc_edge_W, enc_edge_b.reshape(1, 4), hid_edge_We)
    t_flat = t_edge.reshape(n_edges * 8)

    zeros = jnp.zeros((n_nodes, 8), f32)
    parts = _sc_edge_call(n_nodes, n_groups, t_flat, senders, receivers,
                          m_tab, zeros)

    nodes_out, glob_out = pl.pallas_call(
        _node_out_kernel,
        grid=(n_blocks,),
        in_specs=[
            pl.BlockSpec((bn, 8), lambda i: (i, 0)),
            pl.BlockSpec((bn, 8), lambda i: (i, 0)),
            pl.BlockSpec((bn, 8), lambda i: (i + n_blocks, 0)),
            pl.BlockSpec((8, 8), lambda i: (0, 0)),
            pl.BlockSpec((8, 8), lambda i: (0, 0)),
            pl.BlockSpec((1, 8), lambda i: (0, 0)),
            pl.BlockSpec((8, 1), lambda i: (0, 0)),
            pl.BlockSpec((1, 1), lambda i: (0, 0)),
            pl.BlockSpec((8, 1), lambda i: (0, 0)),
            pl.BlockSpec((1, 1), lambda i: (0, 0)),
        ],
        out_specs=[
            pl.BlockSpec((bn, 1), lambda i: (i, 0)),
            pl.BlockSpec((1, 1), lambda i: (0, 0)),
        ],
        out_shape=[
            jax.ShapeDtypeStruct((n_nodes, 1), f32),
            jax.ShapeDtypeStruct((1, 1), f32),
        ],
        scratch_shapes=[pltpu.VMEM((1, 8), f32)],
    )(n_enc, parts, parts, hid_node_Wn, hid_node_Wi,
      hid_node_b.reshape(1, 8), ro_node_W, ro_node_b.reshape(1, 1),
      ro_glob_W, ro_glob_b.reshape(1, 1))

    return nodes_out, glob_out


# speedup vs baseline: 5.0963x; 1.2554x over previous
import functools

import jax
import jax.numpy as jnp
from jax import lax
from jax.experimental import pallas as pl
from jax.experimental.pallas import tpu as pltpu
from jax.experimental.pallas import tpu_sc as plsc

_NC = 2
_NS = 16
_NW = _NC * _NS
_LANES = 16


def _bf(x):
    return x.astype(jnp.bfloat16).astype(jnp.float32)


def _bmm(x, w_ref, k):
    acc = _bf(x[:, 0:1]) * _bf(w_ref[0:1, :])
    for i in range(1, k):
        acc = acc + _bf(x[:, i : i + 1]) * _bf(w_ref[i : i + 1, :])
    return acc


def _node_enc_kernel(nf_ref, wn_ref, bn_ref, ws_ref, be_ref, n_ref, m_ref):
    nf = nf_ref[...]
    n = jnp.maximum(_bmm(nf, wn_ref, 4) + bn_ref[...], 0.0)
    n_ref[...] = n
    m_ref[...] = _bmm(n, ws_ref, 8) + be_ref[...]


def _edge_enc_kernel(ef_ref, w1_ref, b1_ref, we_ref, t_ref):
    ef = ef_ref[...]
    e = jnp.maximum(_bmm(ef, w1_ref, 2) + b1_ref[...], 0.0)
    t_ref[...] = _bmm(e, we_ref, 4)


def _node_out_kernel(n_ref, p0_ref, p1_ref, wn_ref, wi_ref, b_ref,
                     wro_ref, bro_ref, wg_ref, bg_ref,
                     nodes_ref, glob_ref, gacc_ref):
    i = pl.program_id(0)
    nb = pl.num_programs(0)
    n = n_ref[...]
    acc = p0_ref[...] + p1_ref[...]
    n2 = jnp.maximum(_bmm(n, wn_ref, 8) + _bmm(acc, wi_ref, 8) + b_ref[...], 0.0)
    nodes_ref[...] = _bmm(n2, wro_ref, 8) + bro_ref[...]
    bsum = jnp.sum(n2, axis=0, keepdims=True)

    @pl.when(i == 0)
    def _():
        gacc_ref[...] = bsum

    @pl.when(i > 0)
    def _():
        gacc_ref[...] = gacc_ref[...] + bsum

    @pl.when(i == nb - 1)
    def _():
        glob_ref[...] = _bmm(gacc_ref[...], wg_ref, 8) + bg_ref[...]


def _sc_edge_body(n_nodes, n_groups, t_hbm, s_hbm, r_hbm, m_hbm, z_hbm,
                  out_hbm, acc_sp, s_v0, r_v0, t_v0, mg_v0,
                  s_v1, r_v1, t_v1, mg_v1, stage_v,
                  isem0, isem1, gsem0, gsem1):
    cid = lax.axis_index("c")
    sid = lax.axis_index("s")
    wid = sid * _NC + cid

    sr = 800
    nb = (n_nodes // _NS + sr - 1) // sr
    span = sr * nb
    base = pl.multiple_of(jnp.minimum(sid * span, n_nodes - span), 8)

    pltpu.sync_copy(z_hbm.at[pl.ds(0, sr)], stage_v)

    def stage_z(k, c):
        r0 = pl.multiple_of(base + k * sr, 8)
        pltpu.sync_copy(stage_v, acc_sp.at[pl.ds(r0, sr)])
        return c

    lax.fori_loop(0, nb, stage_z, 0)
    plsc.subcore_barrier()

    gper = n_groups // _NW
    rem = n_groups - gper * _NW
    ngroups = gper + jnp.where(wid < rem, 1, 0)
    g0 = wid * gper + jnp.minimum(wid, rem)
    ne = ngroups // 2 * 2

    iota = lax.iota(jnp.int32, _LANES)
    col16 = iota % 8
    rowadd = iota // 8

    bufs = ((s_v0, r_v0, t_v0, mg_v0, isem0, gsem0),
            (s_v1, r_v1, t_v1, mg_v1, isem1, gsem1))

    def start_inputs(k, b):
        s_v, r_v, t_v, _, isem, _ = bufs[b]
        g = g0 + k
        pltpu.async_copy(s_hbm.at[pl.ds(g * 128, 128)], s_v, isem)
        pltpu.async_copy(r_hbm.at[pl.ds(g * 128, 128)], r_v, isem)
        pltpu.async_copy(t_hbm.at[pl.ds(g * 1024, 1024)], t_v, isem)

    def wait_inputs(b):
        s_v, r_v, t_v, _, isem, _ = bufs[b]
        pltpu.make_async_copy(s_hbm.at[pl.ds(0, 128)], s_v, isem).wait()
        pltpu.make_async_copy(r_hbm.at[pl.ds(0, 128)], r_v, isem).wait()
        pltpu.make_async_copy(t_hbm.at[pl.ds(0, 1024)], t_v, isem).wait()

    def start_gather(b):
        s_v, _, _, mg_v, _, gsem = bufs[b]
        pltpu.async_copy(m_hbm.at[s_v], mg_v, gsem)

    def wait_gather(b):
        s_v, _, _, mg_v, _, gsem = bufs[b]
        pltpu.make_async_copy(m_hbm.at[s_v], mg_v, gsem).wait()

    def compute_scatter(b):
        _, r_v, t_v, mg_v, _, _ = bufs[b]

        def compute_body(j, c):
            row16 = 2 * j + rowadd
            tv = t_v[pl.ds(j * 16, 16)]
            mv = plsc.load_gather(mg_v, [row16, col16])
            e2 = jnp.maximum(tv + mv, 0.0)
            plsc.store_scatter(mg_v, [row16, col16], e2)
            return c

        lax.fori_loop(0, 64, compute_body, 0, unroll=8)
        pltpu.sync_copy(mg_v, acc_sp.at[r_v], add=True)

    def phase(k, b):
        wait_gather(b)
        compute_scatter(b)

        @pl.when(k + 2 < ne)
        def _():
            start_inputs(k + 2, b)

        @pl.when(k + 1 < ne)
        def _():
            wait_inputs(1 - b)
            start_gather(1 - b)

    @pl.when(ne > 0)
    def _():
        start_inputs(0, 0)

    @pl.when(ne > 1)
    def _():
        start_inputs(1, 1)

    @pl.when(ne > 0)
    def _():
        wait_inputs(0)
        start_gather(0)

    def pair_body(k2, c):
        phase(k2 * 2, 0)
        phase(k2 * 2 + 1, 1)
        return c

    lax.fori_loop(0, ne // 2, pair_body, 0)

    @pl.when(ngroups > ne)
    def _():
        s_v, r_v, t_v, mg_v, _, gsem = bufs[0]
        g = g0 + ne
        pltpu.sync_copy(s_hbm.at[pl.ds(g * 128, 128)], s_v)
        pltpu.sync_copy(r_hbm.at[pl.ds(g * 128, 128)], r_v)
        pltpu.sync_copy(t_hbm.at[pl.ds(g * 1024, 1024)], t_v)
        pltpu.async_copy(m_hbm.at[s_v], mg_v, gsem).wait()

        def compute_body(j, c):
            row16 = 2 * j + rowadd
            tv = t_v[pl.ds(j * 16, 16)]
            mv = plsc.load_gather(mg_v, [row16, col16])
            e2 = jnp.maximum(tv + mv, 0.0)
            plsc.store_scatter(mg_v, [row16, col16], e2)
            return c

        lax.fori_loop(0, 64, compute_body, 0, unroll=8)
        pltpu.sync_copy(mg_v, acc_sp.at[r_v], add=True)

    plsc.subcore_barrier()

    def unstage(k, c):
        r0 = pl.multiple_of(base + k * sr, 8)
        pltpu.sync_copy(acc_sp.at[pl.ds(r0, sr)], stage_v)
        pltpu.sync_copy(stage_v,
                        out_hbm.at[pl.ds(cid * n_nodes + r0, sr)])
        return c

    lax.fori_loop(0, nb, unstage, 0)


def _sc_edge_call(n_nodes, n_groups, t_edge, senders, receivers, m_tab, zeros):
    mesh = plsc.VectorSubcoreMesh(core_axis_name="c", subcore_axis_name="s",
                                  num_cores=_NC, num_subcores=_NS)
    f32 = jnp.float32
    i32 = jnp.int32
    return pl.kernel(
        functools.partial(_sc_edge_body, n_nodes, n_groups),
        out_type=jax.ShapeDtypeStruct((_NC * n_nodes, 8), f32),
        mesh=mesh,
        scratch_types=[
            pltpu.VMEM_SHARED((n_nodes, 8), f32),
            pltpu.VMEM((128,), i32),
            pltpu.VMEM((128,), i32),
            pltpu.VMEM((1024,), f32),
            pltpu.VMEM((128, 8), f32),
            pltpu.VMEM((128,), i32),
            pltpu.VMEM((128,), i32),
            pltpu.VMEM((1024,), f32),
            pltpu.VMEM((128, 8), f32),
            pltpu.VMEM((800, 8), f32),
            pltpu.SemaphoreType.DMA,
            pltpu.SemaphoreType.DMA,
            pltpu.SemaphoreType.DMA,
            pltpu.SemaphoreType.DMA,
        ],
        compiler_params=pltpu.CompilerParams(
            needs_layout_passes=False, use_tc_tiling_on_sc=False),
    )(t_edge, senders, receivers, m_tab, zeros)


def kernel(node_features, edge_features, senders, receivers, enc_edge_W,
           enc_edge_b, enc_node_W, enc_node_b, hid_edge_We, hid_edge_Ws,
           hid_edge_b, hid_node_Wn, hid_node_Wi, hid_node_b, ro_node_W,
           ro_node_b, ro_glob_W, ro_glob_b):
    n_nodes = node_features.shape[0]
    n_edges = edge_features.shape[0]
    assert n_edges % 128 == 0
    n_groups = n_edges // 128

    f32 = jnp.float32

    bn = 4000 if n_nodes % 4000 == 0 else n_nodes
    n_blocks = n_nodes // bn
    n_enc, m_tab = pl.pallas_call(
        _node_enc_kernel,
        grid=(n_blocks,),
        in_specs=[
            pl.BlockSpec((bn, 4), lambda i: (i, 0)),
            pl.BlockSpec((4, 8), lambda i: (0, 0)),
            pl.BlockSpec((1, 8), lambda i: (0, 0)),
            pl.BlockSpec((8, 8), lambda i: (0, 0)),
            pl.BlockSpec((1, 8), lambda i: (0, 0)),
        ],
        out_specs=[
            pl.BlockSpec((bn, 8), lambda i: (i, 0)),
            pl.BlockSpec((bn, 8), lambda i: (i, 0)),
        ],
        out_shape=[
            jax.ShapeDtypeStruct((n_nodes, 8), f32),
            jax.ShapeDtypeStruct((n_nodes, 8), f32),
        ],
    )(node_features, enc_node_W, enc_node_b.reshape(1, 8), hid_edge_Ws,
      hid_edge_b.reshape(1, 8))

    be = 6400 if n_edges % 6400 == 0 else n_edges
    e_blocks = n_edges // be
    t_edge = pl.pallas_call(
        _edge_enc_kernel,
        grid=(e_blocks,),
        in_specs=[
            pl.BlockSpec((be, 2), lambda i: (i, 0)),
            pl.BlockSpec((2, 4), lambda i: (0, 0)),
            pl.BlockSpec((1, 4), lambda i: (0, 0)),
            pl.BlockSpec((4, 8), lambda i: (0, 0)),
        ],
        out_specs=pl.BlockSpec((be, 8), lambda i: (i, 0)),
        out_shape=jax.ShapeDtypeStruct((n_edges, 8), f32),
    )(edge_features, enc_edge_W, enc_edge_b.reshape(1, 4), hid_edge_We)
    t_flat = t_edge.reshape(n_edges * 8)

    zeros = jnp.zeros((n_nodes, 8), f32)
    parts = _sc_edge_call(n_nodes, n_groups, t_flat, senders, receivers,
                          m_tab, zeros)

    nodes_out, glob_out = pl.pallas_call(
        _node_out_kernel,
        grid=(n_blocks,),
        in_specs=[
            pl.BlockSpec((bn, 8), lambda i: (i, 0)),
            pl.BlockSpec((bn, 8), lambda i: (i, 0)),
            pl.BlockSpec((bn, 8), lambda i: (i + n_blocks, 0)),
            pl.BlockSpec((8, 8), lambda i: (0, 0)),
            pl.BlockSpec((8, 8), lambda i: (0, 0)),
            pl.BlockSpec((1, 8), lambda i: (0, 0)),
            pl.BlockSpec((8, 1), lambda i: (0, 0)),
            pl.BlockSpec((1, 1), lambda i: (0, 0)),
            pl.BlockSpec((8, 1), lambda i: (0, 0)),
            pl.BlockSpec((1, 1), lambda i: (0, 0)),
        ],
        out_specs=[
            pl.BlockSpec((bn, 1), lambda i: (i, 0)),
            pl.BlockSpec((1, 1), lambda i: (0, 0)),
        ],
        out_shape=[
            jax.ShapeDtypeStruct((n_nodes, 1), f32),
            jax.ShapeDtypeStruct((1, 1), f32),
        ],
        scratch_shapes=[pltpu.VMEM((1, 8), f32)],
    )(n_enc, parts, parts, hid_node_Wn, hid_node_Wi,
      hid_node_b.reshape(1, 8), ro_node_W, ro_node_b.reshape(1, 1),
      ro_glob_W, ro_glob_b.reshape(1, 1))

    return nodes_out, glob_out


# speedup vs baseline: 5.7872x; 1.1356x over previous
import functools

import jax
import jax.numpy as jnp
from jax import lax
from jax.experimental import pallas as pl
from jax.experimental.pallas import tpu as pltpu
from jax.experimental.pallas import tpu_sc as plsc

_NC = 2
_NS = 16
_NW = _NC * _NS
_LANES = 16


def _node_enc_kernel(nf_ref, wn_ref, bn_ref, ws_ref, be_ref, n_ref, m_ref):
    nf = nf_ref[...]
    n = jnp.maximum(jnp.dot(nf, wn_ref[...]) + bn_ref[...], 0.0)
    n_ref[...] = n
    m_ref[...] = jnp.dot(n, ws_ref[...]) + be_ref[...]


def _edge_enc_kernel(ef_ref, w1_ref, b1_ref, we_ref, t_ref):
    ef = ef_ref[...]
    e = jnp.maximum(jnp.dot(ef, w1_ref[...]) + b1_ref[...], 0.0)
    t_ref[...] = jnp.dot(e, we_ref[...])


def _node_out_kernel(n_ref, p0_ref, p1_ref, wn_ref, wi_ref, b_ref,
                     wro_ref, bro_ref, wg_ref, bg_ref,
                     nodes_ref, glob_ref, gacc_ref):
    i = pl.program_id(0)
    nb = pl.num_programs(0)
    n = n_ref[...]
    acc = p0_ref[...] + p1_ref[...]
    n2 = jnp.maximum(jnp.dot(n, wn_ref[...]) + jnp.dot(acc, wi_ref[...])
                     + b_ref[...], 0.0)
    nodes_ref[...] = jnp.dot(n2, wro_ref[...]) + bro_ref[...]
    bsum = jnp.sum(n2, axis=0, keepdims=True)

    @pl.when(i == 0)
    def _():
        gacc_ref[...] = bsum

    @pl.when(i > 0)
    def _():
        gacc_ref[...] = gacc_ref[...] + bsum

    @pl.when(i == nb - 1)
    def _():
        glob_ref[...] = jnp.dot(gacc_ref[...], wg_ref[...]) + bg_ref[...]


def _sc_edge_body(n_nodes, n_groups, t_hbm, s_hbm, r_hbm, m_hbm, z_hbm,
                  out_hbm, acc_sp, s_v0, r_v0, t_v0, mg_v0,
                  s_v1, r_v1, t_v1, mg_v1, stage_v,
                  isem0, isem1, gsem0, gsem1):
    cid = lax.axis_index("c")
    sid = lax.axis_index("s")
    wid = sid * _NC + cid

    sr = 800
    nb = (n_nodes // _NS + sr - 1) // sr
    span = sr * nb
    base = pl.multiple_of(jnp.minimum(sid * span, n_nodes - span), 8)

    pltpu.sync_copy(z_hbm.at[pl.ds(0, sr)], stage_v)

    def stage_z(k, c):
        r0 = pl.multiple_of(base + k * sr, 8)
        pltpu.sync_copy(stage_v, acc_sp.at[pl.ds(r0, sr)])
        return c

    lax.fori_loop(0, nb, stage_z, 0)
    plsc.subcore_barrier()

    gper = n_groups // _NW
    rem = n_groups - gper * _NW
    ngroups = gper + jnp.where(wid < rem, 1, 0)
    g0 = wid * gper + jnp.minimum(wid, rem)
    ne = ngroups // 2 * 2

    iota = lax.iota(jnp.int32, _LANES)
    col16 = iota % 8
    rowadd = iota // 8

    bufs = ((s_v0, r_v0, t_v0, mg_v0, isem0, gsem0),
            (s_v1, r_v1, t_v1, mg_v1, isem1, gsem1))

    def start_inputs(k, b):
        s_v, r_v, t_v, _, isem, _ = bufs[b]
        g = g0 + k
        pltpu.async_copy(s_hbm.at[pl.ds(g * 128, 128)], s_v, isem)
        pltpu.async_copy(r_hbm.at[pl.ds(g * 128, 128)], r_v, isem)
        pltpu.async_copy(t_hbm.at[pl.ds(g * 1024, 1024)], t_v, isem)

    def wait_inputs(b):
        s_v, r_v, t_v, _, isem, _ = bufs[b]
        pltpu.make_async_copy(s_hbm.at[pl.ds(0, 128)], s_v, isem).wait()
        pltpu.make_async_copy(r_hbm.at[pl.ds(0, 128)], r_v, isem).wait()
        pltpu.make_async_copy(t_hbm.at[pl.ds(0, 1024)], t_v, isem).wait()

    def start_gather(b):
        s_v, _, _, mg_v, _, gsem = bufs[b]
        pltpu.async_copy(m_hbm.at[s_v], mg_v, gsem)

    def wait_gather(b):
        s_v, _, _, mg_v, _, gsem = bufs[b]
        pltpu.make_async_copy(m_hbm.at[s_v], mg_v, gsem).wait()

    def compute_scatter(b):
        _, r_v, t_v, mg_v, _, _ = bufs[b]

        def compute_body(j, c):
            row16 = 2 * j + rowadd
            tv = t_v[pl.ds(j * 16, 16)]
            mv = plsc.load_gather(mg_v, [row16, col16])
            e2 = jnp.maximum(tv + mv, 0.0)
            plsc.store_scatter(mg_v, [row16, col16], e2)
            return c

        lax.fori_loop(0, 64, compute_body, 0, unroll=8)
        pltpu.sync_copy(mg_v, acc_sp.at[r_v], add=True)

    def phase(k, b):
        wait_gather(b)
        compute_scatter(b)

        @pl.when(k + 2 < ne)
        def _():
            start_inputs(k + 2, b)

        @pl.when(k + 1 < ne)
        def _():
            wait_inputs(1 - b)
            start_gather(1 - b)

    @pl.when(ne > 0)
    def _():
        start_inputs(0, 0)

    @pl.when(ne > 1)
    def _():
        start_inputs(1, 1)

    @pl.when(ne > 0)
    def _():
        wait_inputs(0)
        start_gather(0)

    def pair_body(k2, c):
        phase(k2 * 2, 0)
        phase(k2 * 2 + 1, 1)
        return c

    lax.fori_loop(0, ne // 2, pair_body, 0)

    @pl.when(ngroups > ne)
    def _():
        s_v, r_v, t_v, mg_v, _, gsem = bufs[0]
        g = g0 + ne
        pltpu.sync_copy(s_hbm.at[pl.ds(g * 128, 128)], s_v)
        pltpu.sync_copy(r_hbm.at[pl.ds(g * 128, 128)], r_v)
        pltpu.sync_copy(t_hbm.at[pl.ds(g * 1024, 1024)], t_v)
        pltpu.async_copy(m_hbm.at[s_v], mg_v, gsem).wait()

        def compute_body(j, c):
            row16 = 2 * j + rowadd
            tv = t_v[pl.ds(j * 16, 16)]
            mv = plsc.load_gather(mg_v, [row16, col16])
            e2 = jnp.maximum(tv + mv, 0.0)
            plsc.store_scatter(mg_v, [row16, col16], e2)
            return c

        lax.fori_loop(0, 64, compute_body, 0, unroll=8)
        pltpu.sync_copy(mg_v, acc_sp.at[r_v], add=True)

    plsc.subcore_barrier()

    def unstage(k, c):
        r0 = pl.multiple_of(base + k * sr, 8)
        pltpu.sync_copy(acc_sp.at[pl.ds(r0, sr)], stage_v)
        pltpu.sync_copy(stage_v,
                        out_hbm.at[pl.ds(cid * n_nodes + r0, sr)])
        return c

    lax.fori_loop(0, nb, unstage, 0)


def _sc_edge_call(n_nodes, n_groups, t_edge, senders, receivers, m_tab, zeros):
    mesh = plsc.VectorSubcoreMesh(core_axis_name="c", subcore_axis_name="s",
                                  num_cores=_NC, num_subcores=_NS)
    f32 = jnp.float32
    i32 = jnp.int32
    return pl.kernel(
        functools.partial(_sc_edge_body, n_nodes, n_groups),
        out_type=jax.ShapeDtypeStruct((_NC * n_nodes, 8), f32),
        mesh=mesh,
        scratch_types=[
            pltpu.VMEM_SHARED((n_nodes, 8), f32),
            pltpu.VMEM((128,), i32),
            pltpu.VMEM((128,), i32),
            pltpu.VMEM((1024,), f32),
            pltpu.VMEM((128, 8), f32),
            pltpu.VMEM((128,), i32),
            pltpu.VMEM((128,), i32),
            pltpu.VMEM((1024,), f32),
            pltpu.VMEM((128, 8), f32),
            pltpu.VMEM((800, 8), f32),
            pltpu.SemaphoreType.DMA,
            pltpu.SemaphoreType.DMA,
            pltpu.SemaphoreType.DMA,
            pltpu.SemaphoreType.DMA,
        ],
        compiler_params=pltpu.CompilerParams(
            needs_layout_passes=False, use_tc_tiling_on_sc=False),
    )(t_edge, senders, receivers, m_tab, zeros)


def kernel(node_features, edge_features, senders, receivers, enc_edge_W,
           enc_edge_b, enc_node_W, enc_node_b, hid_edge_We, hid_edge_Ws,
           hid_edge_b, hid_node_Wn, hid_node_Wi, hid_node_b, ro_node_W,
           ro_node_b, ro_glob_W, ro_glob_b):
    n_nodes = node_features.shape[0]
    n_edges = edge_features.shape[0]
    assert n_edges % 128 == 0
    n_groups = n_edges // 128

    f32 = jnp.float32

    bn = 4000 if n_nodes % 4000 == 0 else n_nodes
    n_blocks = n_nodes // bn
    n_enc, m_tab = pl.pallas_call(
        _node_enc_kernel,
        grid=(n_blocks,),
        in_specs=[
            pl.BlockSpec((bn, 4), lambda i: (i, 0)),
            pl.BlockSpec((4, 8), lambda i: (0, 0)),
            pl.BlockSpec((1, 8), lambda i: (0, 0)),
            pl.BlockSpec((8, 8), lambda i: (0, 0)),
            pl.BlockSpec((1, 8), lambda i: (0, 0)),
        ],
        out_specs=[
            pl.BlockSpec((bn, 8), lambda i: (i, 0)),
            pl.BlockSpec((bn, 8), lambda i: (i, 0)),
        ],
        out_shape=[
            jax.ShapeDtypeStruct((n_nodes, 8), f32),
            jax.ShapeDtypeStruct((n_nodes, 8), f32),
        ],
    )(node_features, enc_node_W, enc_node_b.reshape(1, 8), hid_edge_Ws,
      hid_edge_b.reshape(1, 8))

    be = 6400 if n_edges % 6400 == 0 else n_edges
    e_blocks = n_edges // be
    t_edge = pl.pallas_call(
        _edge_enc_kernel,
        grid=(e_blocks,),
        in_specs=[
            pl.BlockSpec((be, 2), lambda i: (i, 0)),
            pl.BlockSpec((2, 4), lambda i: (0, 0)),
            pl.BlockSpec((1, 4), lambda i: (0, 0)),
            pl.BlockSpec((4, 8), lambda i: (0, 0)),
        ],
        out_specs=pl.BlockSpec((be, 8), lambda i: (i, 0)),
        out_shape=jax.ShapeDtypeStruct((n_edges, 8), f32),
    )(edge_features, enc_edge_W, enc_edge_b.reshape(1, 4), hid_edge_We)
    t_flat = t_edge.reshape(n_edges * 8)

    zeros = jnp.zeros((n_nodes, 8), f32)
    parts = _sc_edge_call(n_nodes, n_groups, t_flat, senders, receivers,
                          m_tab, zeros)

    nodes_out, glob_out = pl.pallas_call(
        _node_out_kernel,
        grid=(n_blocks,),
        in_specs=[
            pl.BlockSpec((bn, 8), lambda i: (i, 0)),
            pl.BlockSpec((bn, 8), lambda i: (i, 0)),
            pl.BlockSpec((bn, 8), lambda i: (i + n_blocks, 0)),
            pl.BlockSpec((8, 8), lambda i: (0, 0)),
            pl.BlockSpec((8, 8), lambda i: (0, 0)),
            pl.BlockSpec((1, 8), lambda i: (0, 0)),
            pl.BlockSpec((8, 1), lambda i: (0, 0)),
            pl.BlockSpec((1, 1), lambda i: (0, 0)),
            pl.BlockSpec((8, 1), lambda i: (0, 0)),
            pl.BlockSpec((1, 1), lambda i: (0, 0)),
        ],
        out_specs=[
            pl.BlockSpec((bn, 1), lambda i: (i, 0)),
            pl.BlockSpec((1, 1), lambda i: (0, 0)),
        ],
        out_shape=[
            jax.ShapeDtypeStruct((n_nodes, 1), f32),
            jax.ShapeDtypeStruct((1, 1), f32),
        ],
        scratch_shapes=[pltpu.VMEM((1, 8), f32)],
    )(n_enc, parts, parts, hid_node_Wn, hid_node_Wi,
      hid_node_b.reshape(1, 8), ro_node_W, ro_node_b.reshape(1, 1),
      ro_glob_W, ro_glob_b.reshape(1, 1))

    return nodes_out, glob_out


# speedup vs baseline: 5.9701x; 1.0316x over previous
import functools

import jax
import jax.numpy as jnp
from jax import lax
from jax.experimental import pallas as pl
from jax.experimental.pallas import tpu as pltpu
from jax.experimental.pallas import tpu_sc as plsc

_NC = 2
_NS = 16
_NW = _NC * _NS
_LANES = 16


def _node_enc_kernel(nf_ref, wn_ref, bn_ref, ws_ref, be_ref, n_ref, m_ref):
    nf = nf_ref[...]
    n = jnp.maximum(jnp.dot(nf, wn_ref[...]) + bn_ref[...], 0.0)
    n_ref[...] = n
    m_ref[...] = jnp.dot(n, ws_ref[...]) + be_ref[...]


def _edge_enc_kernel(ef_ref, w1_ref, b1_ref, we_ref, t_ref):
    ef = ef_ref[...]
    e = jnp.maximum(jnp.dot(ef, w1_ref[...]) + b1_ref[...], 0.0)
    t_ref[...] = jnp.dot(e, we_ref[...])


def _node_out_kernel(n_ref, p0_ref, p1_ref, wn_ref, wi_ref, b_ref,
                     wro_ref, bro_ref, wg_ref, bg_ref,
                     nodes_ref, glob_ref, gacc_ref):
    i = pl.program_id(0)
    nb = pl.num_programs(0)
    n = n_ref[...]
    acc = p0_ref[...] + p1_ref[...]
    n2 = jnp.maximum(jnp.dot(n, wn_ref[...]) + jnp.dot(acc, wi_ref[...])
                     + b_ref[...], 0.0)
    nodes_ref[...] = jnp.dot(n2, wro_ref[...]) + bro_ref[...]
    bsum = jnp.sum(n2, axis=0, keepdims=True)

    @pl.when(i == 0)
    def _():
        gacc_ref[...] = bsum

    @pl.when(i > 0)
    def _():
        gacc_ref[...] = gacc_ref[...] + bsum

    @pl.when(i == nb - 1)
    def _():
        glob_ref[...] = jnp.dot(gacc_ref[...], wg_ref[...]) + bg_ref[...]


def _sc_edge_body(n_nodes, n_groups, t_hbm, s_hbm, r_hbm, m_hbm, z_hbm,
                  out_hbm, acc_sp, s_v0, r_v0, t_v0, mg_v0,
                  s_v1, r_v1, t_v1, mg_v1, stage_v,
                  isem0, isem1, gsem0, gsem1):
    cid = lax.axis_index("c")
    sid = lax.axis_index("s")
    wid = sid * _NC + cid

    sr = 800
    nb = (n_nodes // _NS + sr - 1) // sr
    span = sr * nb
    base = pl.multiple_of(jnp.minimum(sid * span, n_nodes - span), 8)

    pltpu.sync_copy(z_hbm.at[pl.ds(0, sr)], stage_v)

    def stage_z(k, c):
        r0 = pl.multiple_of(base + k * sr, 8)
        pltpu.sync_copy(stage_v, acc_sp.at[pl.ds(r0, sr)])
        return c

    lax.fori_loop(0, nb, stage_z, 0)
    plsc.subcore_barrier()

    gper = n_groups // _NW
    rem = n_groups - gper * _NW
    ngroups = gper + jnp.where(wid < rem, 1, 0)
    g0 = wid * gper + jnp.minimum(wid, rem)
    ne = ngroups // 2 * 2

    iota = lax.iota(jnp.int32, _LANES)
    col16 = iota % 8
    rowadd = iota // 8

    bufs = ((s_v0, r_v0, t_v0, mg_v0, isem0, gsem0),
            (s_v1, r_v1, t_v1, mg_v1, isem1, gsem1))

    def start_inputs(k, b):
        s_v, r_v, t_v, _, isem, _ = bufs[b]
        g = g0 + k
        pltpu.async_copy(s_hbm.at[pl.ds(g * 128, 128)], s_v, isem)
        pltpu.async_copy(r_hbm.at[pl.ds(g * 128, 128)], r_v, isem)
        pltpu.async_copy(t_hbm.at[pl.ds(g * 1024, 1024)], t_v, isem)

    def wait_inputs(b):
        s_v, r_v, t_v, _, isem, _ = bufs[b]
        pltpu.make_async_copy(s_hbm.at[pl.ds(0, 128)], s_v, isem).wait()
        pltpu.make_async_copy(r_hbm.at[pl.ds(0, 128)], r_v, isem).wait()
        pltpu.make_async_copy(t_hbm.at[pl.ds(0, 1024)], t_v, isem).wait()

    def start_gather(b):
        s_v, _, _, mg_v, _, gsem = bufs[b]
        pltpu.async_copy(m_hbm.at[s_v], mg_v, gsem)

    def wait_gather(b):
        s_v, _, _, mg_v, _, gsem = bufs[b]
        pltpu.make_async_copy(m_hbm.at[s_v], mg_v, gsem).wait()

    def compute_scatter(b):
        _, r_v, t_v, mg_v, _, _ = bufs[b]

        def compute_body(j, c):
            row16 = 2 * j + rowadd
            tv = t_v[pl.ds(j * 16, 16)]
            mv = plsc.load_gather(mg_v, [row16, col16])
            e2 = jnp.maximum(tv + mv, 0.0)
            plsc.store_scatter(mg_v, [row16, col16], e2)
            return c

        lax.fori_loop(0, 64, compute_body, 0, unroll=8)
        pltpu.sync_copy(mg_v, acc_sp.at[r_v], add=True)

    def phase(k, b):
        wait_gather(b)

        @pl.when(k + 1 < ne)
        def _():
            wait_inputs(1 - b)
            start_gather(1 - b)

        compute_scatter(b)

        @pl.when(k + 2 < ne)
        def _():
            start_inputs(k + 2, b)

    @pl.when(ne > 0)
    def _():
        start_inputs(0, 0)

    @pl.when(ne > 1)
    def _():
        start_inputs(1, 1)

    @pl.when(ne > 0)
    def _():
        wait_inputs(0)
        start_gather(0)

    def pair_body(k2, c):
        phase(k2 * 2, 0)
        phase(k2 * 2 + 1, 1)
        return c

    lax.fori_loop(0, ne // 2, pair_body, 0)

    @pl.when(ngroups > ne)
    def _():
        s_v, r_v, t_v, mg_v, _, gsem = bufs[0]
        g = g0 + ne
        pltpu.sync_copy(s_hbm.at[pl.ds(g * 128, 128)], s_v)
        pltpu.sync_copy(r_hbm.at[pl.ds(g * 128, 128)], r_v)
        pltpu.sync_copy(t_hbm.at[pl.ds(g * 1024, 1024)], t_v)
        pltpu.async_copy(m_hbm.at[s_v], mg_v, gsem).wait()

        def compute_body(j, c):
            row16 = 2 * j + rowadd
            tv = t_v[pl.ds(j * 16, 16)]
            mv = plsc.load_gather(mg_v, [row16, col16])
            e2 = jnp.maximum(tv + mv, 0.0)
            plsc.store_scatter(mg_v, [row16, col16], e2)
            return c

        lax.fori_loop(0, 64, compute_body, 0, unroll=8)
        pltpu.sync_copy(mg_v, acc_sp.at[r_v], add=True)

    plsc.subcore_barrier()

    def unstage(k, c):
        r0 = pl.multiple_of(base + k * sr, 8)
        pltpu.sync_copy(acc_sp.at[pl.ds(r0, sr)], stage_v)
        pltpu.sync_copy(stage_v,
                        out_hbm.at[pl.ds(cid * n_nodes + r0, sr)])
        return c

    lax.fori_loop(0, nb, unstage, 0)


def _sc_edge_call(n_nodes, n_groups, t_edge, senders, receivers, m_tab, zeros):
    mesh = plsc.VectorSubcoreMesh(core_axis_name="c", subcore_axis_name="s",
                                  num_cores=_NC, num_subcores=_NS)
    f32 = jnp.float32
    i32 = jnp.int32
    return pl.kernel(
        functools.partial(_sc_edge_body, n_nodes, n_groups),
        out_type=jax.ShapeDtypeStruct((_NC * n_nodes, 8), f32),
        mesh=mesh,
        scratch_types=[
            pltpu.VMEM_SHARED((n_nodes, 8), f32),
            pltpu.VMEM((128,), i32),
            pltpu.VMEM((128,), i32),
            pltpu.VMEM((1024,), f32),
            pltpu.VMEM((128, 8), f32),
            pltpu.VMEM((128,), i32),
            pltpu.VMEM((128,), i32),
            pltpu.VMEM((1024,), f32),
            pltpu.VMEM((128, 8), f32),
            pltpu.VMEM((800, 8), f32),
            pltpu.SemaphoreType.DMA,
            pltpu.SemaphoreType.DMA,
            pltpu.SemaphoreType.DMA,
            pltpu.SemaphoreType.DMA,
        ],
        compiler_params=pltpu.CompilerParams(
            needs_layout_passes=False, use_tc_tiling_on_sc=False),
    )(t_edge, senders, receivers, m_tab, zeros)


def kernel(node_features, edge_features, senders, receivers, enc_edge_W,
           enc_edge_b, enc_node_W, enc_node_b, hid_edge_We, hid_edge_Ws,
           hid_edge_b, hid_node_Wn, hid_node_Wi, hid_node_b, ro_node_W,
           ro_node_b, ro_glob_W, ro_glob_b):
    n_nodes = node_features.shape[0]
    n_edges = edge_features.shape[0]
    assert n_edges % 128 == 0
    n_groups = n_edges // 128

    f32 = jnp.float32

    bn = 4000 if n_nodes % 4000 == 0 else n_nodes
    n_blocks = n_nodes // bn
    n_enc, m_tab = pl.pallas_call(
        _node_enc_kernel,
        grid=(n_blocks,),
        in_specs=[
            pl.BlockSpec((bn, 4), lambda i: (i, 0)),
            pl.BlockSpec((4, 8), lambda i: (0, 0)),
            pl.BlockSpec((1, 8), lambda i: (0, 0)),
            pl.BlockSpec((8, 8), lambda i: (0, 0)),
            pl.BlockSpec((1, 8), lambda i: (0, 0)),
        ],
        out_specs=[
            pl.BlockSpec((bn, 8), lambda i: (i, 0)),
            pl.BlockSpec((bn, 8), lambda i: (i, 0)),
        ],
        out_shape=[
            jax.ShapeDtypeStruct((n_nodes, 8), f32),
            jax.ShapeDtypeStruct((n_nodes, 8), f32),
        ],
    )(node_features, enc_node_W, enc_node_b.reshape(1, 8), hid_edge_Ws,
      hid_edge_b.reshape(1, 8))

    be = 6400 if n_edges % 6400 == 0 else n_edges
    e_blocks = n_edges // be
    t_edge = pl.pallas_call(
        _edge_enc_kernel,
        grid=(e_blocks,),
        in_specs=[
            pl.BlockSpec((be, 2), lambda i: (i, 0)),
            pl.BlockSpec((2, 4), lambda i: (0, 0)),
            pl.BlockSpec((1, 4), lambda i: (0, 0)),
            pl.BlockSpec((4, 8), lambda i: (0, 0)),
        ],
        out_specs=pl.BlockSpec((be, 8), lambda i: (i, 0)),
        out_shape=jax.ShapeDtypeStruct((n_edges, 8), f32),
    )(edge_features, enc_edge_W, enc_edge_b.reshape(1, 4), hid_edge_We)
    t_flat = t_edge.reshape(n_edges * 8)

    zeros = jnp.zeros((n_nodes, 8), f32)
    parts = _sc_edge_call(n_nodes, n_groups, t_flat, senders, receivers,
                          m_tab, zeros)

    nodes_out, glob_out = pl.pallas_call(
        _node_out_kernel,
        grid=(n_blocks,),
        in_specs=[
            pl.BlockSpec((bn, 8), lambda i: (i, 0)),
            pl.BlockSpec((bn, 8), lambda i: (i, 0)),
            pl.BlockSpec((bn, 8), lambda i: (i + n_blocks, 0)),
            pl.BlockSpec((8, 8), lambda i: (0, 0)),
            pl.BlockSpec((8, 8), lambda i: (0, 0)),
            pl.BlockSpec((1, 8), lambda i: (0, 0)),
            pl.BlockSpec((8, 1), lambda i: (0, 0)),
            pl.BlockSpec((1, 1), lambda i: (0, 0)),
            pl.BlockSpec((8, 1), lambda i: (0, 0)),
            pl.BlockSpec((1, 1), lambda i: (0, 0)),
        ],
        out_specs=[
            pl.BlockSpec((bn, 1), lambda i: (i, 0)),
            pl.BlockSpec((1, 1), lambda i: (0, 0)),
        ],
        out_shape=[
            jax.ShapeDtypeStruct((n_nodes, 1), f32),
            jax.ShapeDtypeStruct((1, 1), f32),
        ],
        scratch_shapes=[pltpu.VMEM((1, 8), f32)],
    )(n_enc, parts, parts, hid_node_Wn, hid_node_Wi,
      hid_node_b.reshape(1, 8), ro_node_W, ro_node_b.reshape(1, 1),
      ro_glob_W, ro_glob_b.reshape(1, 1))

    return nodes_out, glob_out
